# Initial kernel scaffold; baseline (speedup 1.0000x reference)
import jax
import jax.numpy as jnp
from jax.experimental import pallas as pl


def kernel(img, reg_lab):
    raise NotImplementedError("write your pallas kernel here")


# speedup vs baseline: 8.5182x; 8.5182x over previous
import functools

import jax
import jax.numpy as jnp
from jax import lax
from jax.experimental import pallas as pl
from jax.experimental.pallas import tpu as pltpu
from jax.experimental.pallas import tpu_sc as plsc

_NB_SEGS = 1024
_BINS = 25
_C = 3
_H = 512
_W = 512
_N = _H * _W
_NC = 2
_NS = 16
_NW = _NC * _NS
_P = _N // _NW
_ROW = _C * _BINS
_HIST = _NB_SEGS * _ROW
_L = 16


def _sc_partial_hists(img_flat, reg_flat):
    mesh = plsc.VectorSubcoreMesh(core_axis_name="c", subcore_axis_name="s")

    @functools.partial(
        pl.kernel,
        out_type=jax.ShapeDtypeStruct((_NW, _HIST), jnp.float32),
        mesh=mesh,
        scratch_types=[
            pltpu.VMEM((_HIST,), jnp.float32),
            pltpu.VMEM((_P,), jnp.int32),
            pltpu.VMEM((_C * _P,), jnp.float32),
            pltpu.SemaphoreType.DMA,
        ],
        compiler_params=pltpu.CompilerParams(needs_layout_passes=False),
    )
    def sc_hist(img_hbm, reg_hbm, out_hbm, hist_v, reg_v, img_v, sem):
        wid = lax.axis_index("s") * _NC + lax.axis_index("c")
        base = wid * _P

        reg_cp = pltpu.async_copy(reg_hbm.at[pl.ds(base, _P)], reg_v, sem)
        img_cps = [
            pltpu.async_copy(
                img_hbm.at[pl.ds(c * _N + base, _P)],
                img_v.at[pl.ds(c * _P, _P)],
                sem,
            )
            for c in range(_C)
        ]

        @pl.loop(0, _HIST, step=_L)
        def _(i):
            hist_v[pl.ds(i, _L)] = jnp.zeros((_L,), jnp.float32)

        reg_cp.wait()
        for cp in img_cps:
            cp.wait()

        ones = jnp.ones((_L,), jnp.float32)
        for c in range(_C):
            coff = c * _BINS

            @pl.loop(0, _P, step=_L)
            def _(i, c=c, coff=coff):
                r = reg_v[pl.ds(i, _L)]
                x = img_v[pl.ds(c * _P + i, _L)]
                b = (x * float(_BINS - 1)).astype(jnp.int32)
                b = jnp.minimum(jnp.maximum(b, 0), _BINS - 1)
                z = r * _ROW + (b + coff)
                plsc.addupdate_scatter(hist_v, [z], ones)

        pltpu.sync_copy(hist_v, out_hbm.at[wid])

    return sc_hist(img_flat, reg_flat)


def _tc_merge(partials):
    def body(p_ref, hist_ref, rs_ref):
        s = jnp.sum(p_ref[...], axis=0)
        tot = jnp.sum(s, axis=1, keepdims=True)
        hist_ref[...] = s / tot
        rs_ref[...] = tot / 3.0

    return pl.pallas_call(
        body,
        out_shape=(
            jax.ShapeDtypeStruct((_NB_SEGS, _ROW), jnp.float32),
            jax.ShapeDtypeStruct((_NB_SEGS, 1), jnp.float32),
        ),
    )(partials)


def kernel(img, reg_lab):
    img_flat = img.reshape(_C * _N)
    reg_flat = reg_lab.reshape(_N)
    partials = _sc_partial_hists(img_flat, reg_flat)
    hist2d, rs = _tc_merge(partials.reshape(_NW, _NB_SEGS, _ROW))
    return hist2d.reshape(_NB_SEGS, _C, _BINS), rs.reshape(_NB_SEGS)
